# baseline (speedup 1.0000x reference)
import functools

import jax
import jax.numpy as jnp
from jax import lax
from jax.experimental import pallas as pl
from jax.experimental.pallas import tpu as pltpu
from jax.experimental.pallas import tpu_sc as plsc

N_NODES = 10000
N_EDGES = 320000
D = 128

NC = 2
NS = 16
NW = NC * NS
K = 125
CH = 80
NP = 2
HCH = CH // NP
NPAD = 10240
DEGW = 16
RPW = NPAD // NS
BR = 200

_mesh = lambda: plsc.VectorSubcoreMesh(core_axis_name="c", subcore_axis_name="s")


def _deg_body(dst_hbm, zeros_hbm, ones_hbm, deg_out, dst_v, ones_v, deg_sem,
              deg_sh):
    c = lax.axis_index("c")
    s = lax.axis_index("s")
    w = s * NC + c
    pltpu.sync_copy(dst_hbm.at[w], dst_v)
    pltpu.sync_copy(ones_hbm, ones_v)
    pltpu.sync_copy(zeros_hbm.at[pl.ds(s * RPW, RPW)],
                    deg_sh.at[pl.ds(s * RPW, RPW)])
    plsc.subcore_barrier()

    def issue(j, carry):
        pltpu.async_copy(ones_v, deg_sh.at[dst_v.at[j]], deg_sem, add=True)
        return carry

    lax.fori_loop(0, CH, issue, 0)

    def drain(j, carry):
        pltpu.make_async_copy(ones_v, deg_sh.at[dst_v.at[0]], deg_sem).wait()
        return carry

    lax.fori_loop(0, CH, drain, 0)
    plsc.subcore_barrier()
    pltpu.sync_copy(deg_sh.at[pl.ds(s * RPW, RPW)],
                    deg_out.at[c, pl.ds(s * RPW, RPW)])


@jax.jit
def _deg_kernel(dst3, zeros_deg, ones_k):
    return pl.kernel(
        _deg_body,
        out_type=jax.ShapeDtypeStruct((NC, NPAD, DEGW), jnp.float32),
        mesh=_mesh(),
        scratch_types=[
            pltpu.VMEM((CH, K), jnp.int32),
            pltpu.VMEM((K, DEGW), jnp.float32),
            pltpu.SemaphoreType.DMA,
            pltpu.VMEM_SHARED((NPAD, DEGW), jnp.float32),
        ],
    )(dst3, zeros_deg, ones_k)


def _scale_body(x_ref, dp_a_ref, dp_b_ref, y_ref, dis_ref):
    d = dp_a_ref[0] + dp_b_ref[0]
    r = lax.rsqrt(d)
    y_ref[...] = x_ref[...] * r[:, 0:1]
    dis_ref[...] = jnp.where(d > 0.0, r, 0.0)


@jax.jit
def _scale_kernel(x, deg_p):
    return pl.pallas_call(
        _scale_body,
        grid=(N_NODES // BR,),
        in_specs=[
            pl.BlockSpec((BR, D), lambda i: (i, 0)),
            pl.BlockSpec((1, BR, DEGW), lambda i: (0, i, 0)),
            pl.BlockSpec((1, BR, DEGW), lambda i: (1, i, 0)),
        ],
        out_specs=[
            pl.BlockSpec((BR, D), lambda i: (i, 0)),
            pl.BlockSpec((BR, DEGW), lambda i: (i, 0)),
        ],
        out_shape=[
            jax.ShapeDtypeStruct((N_NODES, D), jnp.float32),
            jax.ShapeDtypeStruct((N_NODES, DEGW), jnp.float32),
        ],
    )(x, deg_p, deg_p)


def _agg_body(y_hbm, src_hbm, dst_hbm, zeros_hbm, acc_out,
              src_v, dst_v, rows0, rows1, g0, g1, s0, s1, acc_sh):
    c = lax.axis_index("c")
    s = lax.axis_index("s")
    w = s * NC + c
    pltpu.sync_copy(zeros_hbm.at[pl.ds(s * RPW, RPW)],
                    acc_sh.at[pl.ds(s * RPW, RPW)])
    plsc.subcore_barrier()

    for p in range(NP):
        pltpu.sync_copy(src_hbm.at[w, pl.ds(p * HCH, HCH)], src_v)
        pltpu.sync_copy(dst_hbm.at[w, pl.ds(p * HCH, HCH)], dst_v)
        pltpu.async_copy(y_hbm.at[src_v.at[0]], rows0, g0)
        pltpu.async_copy(y_hbm.at[src_v.at[1]], rows1, g1)

        def body(i, carry):
            j0 = 2 * i
            j1 = j0 + 1
            pltpu.make_async_copy(y_hbm.at[src_v.at[j0]], rows0, g0).wait()
            pltpu.async_copy(rows0, acc_sh.at[dst_v.at[j0]], s0, add=True)
            pltpu.make_async_copy(y_hbm.at[src_v.at[j1]], rows1, g1).wait()
            pltpu.async_copy(rows1, acc_sh.at[dst_v.at[j1]], s1, add=True)

            @pl.when(i < HCH // 2 - 1)
            def _():
                pltpu.make_async_copy(rows0, acc_sh.at[dst_v.at[0]], s0).wait()
                pltpu.async_copy(y_hbm.at[src_v.at[j0 + 2]], rows0, g0)
                pltpu.make_async_copy(rows1, acc_sh.at[dst_v.at[0]], s1).wait()
                pltpu.async_copy(y_hbm.at[src_v.at[j1 + 2]], rows1, g1)

            return carry

        lax.fori_loop(0, HCH // 2, body, 0)
        pltpu.make_async_copy(rows0, acc_sh.at[dst_v.at[0]], s0).wait()
        pltpu.make_async_copy(rows1, acc_sh.at[dst_v.at[0]], s1).wait()
    plsc.subcore_barrier()
    pltpu.sync_copy(acc_sh.at[pl.ds(s * RPW, RPW)],
                    acc_out.at[c, pl.ds(s * RPW, RPW)])


@jax.jit
def _agg_kernel(y, src3, dst3, zeros_big):
    return pl.kernel(
        _agg_body,
        out_type=jax.ShapeDtypeStruct((NC, NPAD, D), jnp.float32),
        mesh=_mesh(),
        scratch_types=[
            pltpu.VMEM((HCH, K), jnp.int32),
            pltpu.VMEM((HCH, K), jnp.int32),
            pltpu.VMEM((K, D), jnp.float32),
            pltpu.VMEM((K, D), jnp.float32),
            pltpu.SemaphoreType.DMA,
            pltpu.SemaphoreType.DMA,
            pltpu.SemaphoreType.DMA,
            pltpu.SemaphoreType.DMA,
            pltpu.VMEM_SHARED((NPAD, D), jnp.float32),
        ],
    )(y, src3, dst3, zeros_big)


def _final_body(a_ref, b_ref, dis_ref, out_ref):
    acc = a_ref[0] + b_ref[0]
    out_ref[...] = jnp.maximum(acc * dis_ref[:, 0:1], 0.0)


@jax.jit
def _final_kernel(acc, dis):
    return pl.pallas_call(
        _final_body,
        grid=(N_NODES // BR,),
        in_specs=[
            pl.BlockSpec((1, BR, D), lambda i: (0, i, 0)),
            pl.BlockSpec((1, BR, D), lambda i: (1, i, 0)),
            pl.BlockSpec((BR, DEGW), lambda i: (i, 0)),
        ],
        out_specs=pl.BlockSpec((BR, D), lambda i: (i, 0)),
        out_shape=jax.ShapeDtypeStruct((N_NODES, D), jnp.float32),
    )(acc, acc, dis)


@jax.jit
def kernel(x, edge_index, line_graph_val):
    src3 = edge_index[0].astype(jnp.int32).reshape(NW, CH, K)
    dst3 = edge_index[1].astype(jnp.int32).reshape(NW, CH, K)

    zeros_deg = jnp.zeros((NPAD, DEGW), jnp.float32)
    ones_k = jnp.ones((K, DEGW), jnp.float32)
    zeros_big = jnp.zeros((NPAD, D), jnp.float32)

    deg_p = _deg_kernel(dst3, zeros_deg, ones_k)
    y, dis = _scale_kernel(x, deg_p)
    acc = _agg_kernel(y, src3, dst3, zeros_big)
    return _final_kernel(acc, dis)


# speedup vs baseline: 1.1545x; 1.1545x over previous
import functools

import jax
import jax.numpy as jnp
from jax import lax
from jax.experimental import pallas as pl
from jax.experimental.pallas import tpu as pltpu
from jax.experimental.pallas import tpu_sc as plsc

N_NODES = 10000
N_EDGES = 320000
D = 128

NC = 2
NS = 16
NW = NC * NS
K = 125
CH = 80
NP = 2
HCH = CH // NP
NPAD = 10240
DEGW = 16
RPW = NPAD // NS
BR = 200

_mesh = lambda: plsc.VectorSubcoreMesh(core_axis_name="c", subcore_axis_name="s")


def _deg_body(dst_hbm, zeros_hbm, ones_hbm, deg_out, dst_v, ones_v, deg_sem,
              deg_sh):
    c = lax.axis_index("c")
    s = lax.axis_index("s")
    w = s * NC + c
    pltpu.sync_copy(dst_hbm.at[w], dst_v)
    pltpu.sync_copy(ones_hbm, ones_v)
    pltpu.sync_copy(zeros_hbm.at[pl.ds(s * RPW, RPW)],
                    deg_sh.at[pl.ds(s * RPW, RPW)])
    plsc.subcore_barrier()

    def issue(j, carry):
        pltpu.async_copy(ones_v, deg_sh.at[dst_v.at[j]], deg_sem, add=True)
        return carry

    lax.fori_loop(0, CH, issue, 0)

    def drain(j, carry):
        pltpu.make_async_copy(ones_v, deg_sh.at[dst_v.at[0]], deg_sem).wait()
        return carry

    lax.fori_loop(0, CH, drain, 0)
    plsc.subcore_barrier()
    pltpu.sync_copy(deg_sh.at[pl.ds(s * RPW, RPW)],
                    deg_out.at[c, pl.ds(s * RPW, RPW)])


@jax.jit
def _deg_kernel(dst3, zeros_deg, ones_k):
    return pl.kernel(
        _deg_body,
        out_type=jax.ShapeDtypeStruct((NC, NPAD, DEGW), jnp.float32),
        mesh=_mesh(),
        scratch_types=[
            pltpu.VMEM((CH, K), jnp.int32),
            pltpu.VMEM((K, DEGW), jnp.float32),
            pltpu.SemaphoreType.DMA,
            pltpu.VMEM_SHARED((NPAD, DEGW), jnp.float32),
        ],
    )(dst3, zeros_deg, ones_k)


def _scale_body(x_ref, dp_a_ref, dp_b_ref, y_ref, dis_ref):
    d = dp_a_ref[0] + dp_b_ref[0]
    r = lax.rsqrt(d)
    y_ref[...] = x_ref[...] * r[:, 0:1]
    dis_ref[...] = jnp.where(d > 0.0, r, 0.0)


@jax.jit
def _scale_kernel(x, deg_p):
    return pl.pallas_call(
        _scale_body,
        grid=(N_NODES // BR,),
        in_specs=[
            pl.BlockSpec((BR, D), lambda i: (i, 0)),
            pl.BlockSpec((1, BR, DEGW), lambda i: (0, i, 0)),
            pl.BlockSpec((1, BR, DEGW), lambda i: (1, i, 0)),
        ],
        out_specs=[
            pl.BlockSpec((BR, D), lambda i: (i, 0)),
            pl.BlockSpec((BR, DEGW), lambda i: (i, 0)),
        ],
        out_shape=[
            jax.ShapeDtypeStruct((N_NODES, D), jnp.float32),
            jax.ShapeDtypeStruct((N_NODES, DEGW), jnp.float32),
        ],
    )(x, deg_p, deg_p)


def _agg_body(y_hbm, src_hbm, dst_hbm, zeros_hbm, acc_out,
              src_v, dst_v, rows0, rows1, g0, g1, s0, s1, acc_sh):
    c = lax.axis_index("c")
    s = lax.axis_index("s")
    w = s * NC + c
    pltpu.sync_copy(zeros_hbm.at[pl.ds(s * RPW, RPW)],
                    acc_sh.at[pl.ds(s * RPW, RPW)])
    plsc.subcore_barrier()

    for p in range(NP):
        pltpu.sync_copy(src_hbm.at[w, pl.ds(p * HCH, HCH)], src_v)
        pltpu.sync_copy(dst_hbm.at[w, pl.ds(p * HCH, HCH)], dst_v)
        pltpu.async_copy(y_hbm.at[src_v.at[0]], rows0, g0)

        def body(i, carry):
            j0 = 2 * i
            j1 = j0 + 1
            cp1 = pltpu.async_copy(y_hbm.at[src_v.at[j1]], rows1, g1)
            pltpu.make_async_copy(y_hbm.at[src_v.at[j0]], rows0, g0).wait()
            pltpu.sync_copy(rows0, acc_sh.at[dst_v.at[j0]], add=True)

            @pl.when(i < HCH // 2 - 1)
            def _():
                pltpu.async_copy(y_hbm.at[src_v.at[j0 + 2]], rows0, g0)

            cp1.wait()
            pltpu.sync_copy(rows1, acc_sh.at[dst_v.at[j1]], add=True)
            return carry

        lax.fori_loop(0, HCH // 2, body, 0)
    plsc.subcore_barrier()
    pltpu.sync_copy(acc_sh.at[pl.ds(s * RPW, RPW)],
                    acc_out.at[c, pl.ds(s * RPW, RPW)])


@jax.jit
def _agg_kernel(y, src3, dst3, zeros_big):
    return pl.kernel(
        _agg_body,
        out_type=jax.ShapeDtypeStruct((NC, NPAD, D), jnp.float32),
        mesh=_mesh(),
        scratch_types=[
            pltpu.VMEM((HCH, K), jnp.int32),
            pltpu.VMEM((HCH, K), jnp.int32),
            pltpu.VMEM((K, D), jnp.float32),
            pltpu.VMEM((K, D), jnp.float32),
            pltpu.SemaphoreType.DMA,
            pltpu.SemaphoreType.DMA,
            pltpu.SemaphoreType.DMA,
            pltpu.SemaphoreType.DMA,
            pltpu.VMEM_SHARED((NPAD, D), jnp.float32),
        ],
    )(y, src3, dst3, zeros_big)


def _final_body(a_ref, b_ref, dis_ref, out_ref):
    acc = a_ref[0] + b_ref[0]
    out_ref[...] = jnp.maximum(acc * dis_ref[:, 0:1], 0.0)


@jax.jit
def _final_kernel(acc, dis):
    return pl.pallas_call(
        _final_body,
        grid=(N_NODES // BR,),
        in_specs=[
            pl.BlockSpec((1, BR, D), lambda i: (0, i, 0)),
            pl.BlockSpec((1, BR, D), lambda i: (1, i, 0)),
            pl.BlockSpec((BR, DEGW), lambda i: (i, 0)),
        ],
        out_specs=pl.BlockSpec((BR, D), lambda i: (i, 0)),
        out_shape=jax.ShapeDtypeStruct((N_NODES, D), jnp.float32),
    )(acc, acc, dis)


@jax.jit
def kernel(x, edge_index, line_graph_val):
    src3 = edge_index[0].astype(jnp.int32).reshape(NW, CH, K)
    dst3 = edge_index[1].astype(jnp.int32).reshape(NW, CH, K)

    zeros_deg = jnp.zeros((NPAD, DEGW), jnp.float32)
    ones_k = jnp.ones((K, DEGW), jnp.float32)
    zeros_big = jnp.zeros((NPAD, D), jnp.float32)

    deg_p = _deg_kernel(dst3, zeros_deg, ones_k)
    y, dis = _scale_kernel(x, deg_p)
    acc = _agg_kernel(y, src3, dst3, zeros_big)
    return _final_kernel(acc, dis)


# speedup vs baseline: 1.2493x; 1.0821x over previous
import functools

import jax
import jax.numpy as jnp
from jax import lax
from jax.experimental import pallas as pl
from jax.experimental.pallas import tpu as pltpu
from jax.experimental.pallas import tpu_sc as plsc

N_NODES = 10000
N_EDGES = 320000
D = 128

NC = 2
NS = 16
NW = NC * NS
K = 80
CH = 125
NP = 5
HCH = CH // NP
NBUF = 3
NPAD = 10240
DEGW = 16
RPW = NPAD // NS
BR = 200

_mesh = lambda: plsc.VectorSubcoreMesh(core_axis_name="c", subcore_axis_name="s")


def _deg_body(dst_hbm, zeros_hbm, ones_hbm, deg_out, dst_v, ones_v, deg_sem,
              deg_sh):
    c = lax.axis_index("c")
    s = lax.axis_index("s")
    w = s * NC + c
    pltpu.sync_copy(dst_hbm.at[w], dst_v)
    pltpu.sync_copy(ones_hbm, ones_v)
    pltpu.sync_copy(zeros_hbm.at[pl.ds(s * RPW, RPW)],
                    deg_sh.at[pl.ds(s * RPW, RPW)])
    plsc.subcore_barrier()

    def issue(j, carry):
        p = j // HCH
        q = j % HCH
        pltpu.async_copy(ones_v, deg_sh.at[dst_v.at[p, q]], deg_sem, add=True)
        return carry

    lax.fori_loop(0, CH, issue, 0)

    def drain(j, carry):
        pltpu.make_async_copy(ones_v, deg_sh.at[dst_v.at[0, 0]],
                              deg_sem).wait()
        return carry

    lax.fori_loop(0, CH, drain, 0)
    plsc.subcore_barrier()
    pltpu.sync_copy(deg_sh.at[pl.ds(s * RPW, RPW)],
                    deg_out.at[c, pl.ds(s * RPW, RPW)])


@jax.jit
def _deg_kernel(dst4, zeros_deg, ones_k):
    return pl.kernel(
        _deg_body,
        out_type=jax.ShapeDtypeStruct((NC, NPAD, DEGW), jnp.float32),
        mesh=_mesh(),
        scratch_types=[
            pltpu.VMEM((NP, HCH, K), jnp.int32),
            pltpu.VMEM((K, DEGW), jnp.float32),
            pltpu.SemaphoreType.DMA,
            pltpu.VMEM_SHARED((NPAD, DEGW), jnp.float32),
        ],
    )(dst4, zeros_deg, ones_k)


def _scale_body(x_ref, dp_a_ref, dp_b_ref, y_ref, dis_ref):
    d = dp_a_ref[0] + dp_b_ref[0]
    r = lax.rsqrt(d)
    y_ref[...] = x_ref[...] * r[:, 0:1]
    dis_ref[...] = jnp.where(d > 0.0, r, 0.0)


@jax.jit
def _scale_kernel(x, deg_p):
    return pl.pallas_call(
        _scale_body,
        grid=(N_NODES // BR,),
        in_specs=[
            pl.BlockSpec((BR, D), lambda i: (i, 0)),
            pl.BlockSpec((1, BR, DEGW), lambda i: (0, i, 0)),
            pl.BlockSpec((1, BR, DEGW), lambda i: (1, i, 0)),
        ],
        out_specs=[
            pl.BlockSpec((BR, D), lambda i: (i, 0)),
            pl.BlockSpec((BR, DEGW), lambda i: (i, 0)),
        ],
        out_shape=[
            jax.ShapeDtypeStruct((N_NODES, D), jnp.float32),
            jax.ShapeDtypeStruct((N_NODES, DEGW), jnp.float32),
        ],
    )(x, deg_p, deg_p)


def _agg_body(y_hbm, src_hbm, dst_hbm, acc_out,
              src0, src1, dst0, dst1, rows0, rows1, rows2,
              g0, g1, g2, isem, acc_sh):
    c = lax.axis_index("c")
    s = lax.axis_index("s")
    w = s * NC + c
    rows = [rows0, rows1, rows2]
    gsem = [g0, g1, g2]
    srcb = [src0, src1]
    dstb = [dst0, dst1]

    def zrow(r, carry):
        for l in range(D // 16):
            rows0[r, pl.ds(l * 16, 16)] = jnp.zeros((16,), jnp.float32)
        return carry

    lax.fori_loop(0, K, zrow, 0)
    base = s * RPW
    for q in range(RPW // K):
        pltpu.sync_copy(rows0, acc_sh.at[pl.ds(base + q * K, K)])
    plsc.subcore_barrier()

    pltpu.sync_copy(src_hbm.at[w, 0], src0)
    pltpu.sync_copy(dst_hbm.at[w, 0], dst0)
    for j in range(NBUF):
        pltpu.async_copy(y_hbm.at[src0.at[j]], rows[j], gsem[j])

    TOT = NP * HCH
    for j in range(TOT):
        p, q, b = j // HCH, j % HCH, j % NBUF
        if q == 0 and p + 1 < NP:
            pltpu.async_copy(src_hbm.at[w, p + 1], srcb[(p + 1) % 2], isem)
            pltpu.async_copy(dst_hbm.at[w, p + 1], dstb[(p + 1) % 2], isem)
        if q == HCH - NBUF and p + 1 < NP:
            pltpu.make_async_copy(src_hbm.at[w, 0],
                                  srcb[(p + 1) % 2], isem).wait()
            pltpu.make_async_copy(dst_hbm.at[w, 0],
                                  dstb[(p + 1) % 2], isem).wait()
        pltpu.make_async_copy(y_hbm.at[src0.at[0]], rows[b], gsem[b]).wait()
        pltpu.sync_copy(rows[b], acc_sh.at[dstb[p % 2].at[q]], add=True)
        jn = j + NBUF
        if jn < TOT:
            pn, qn = jn // HCH, jn % HCH
            pltpu.async_copy(y_hbm.at[srcb[pn % 2].at[qn]], rows[b], gsem[b])

    plsc.subcore_barrier()
    pltpu.sync_copy(acc_sh.at[pl.ds(base, RPW)],
                    acc_out.at[c, pl.ds(base, RPW)])


@jax.jit
def _agg_kernel(y, src4, dst4):
    return pl.kernel(
        _agg_body,
        out_type=jax.ShapeDtypeStruct((NC, NPAD, D), jnp.float32),
        mesh=_mesh(),
        scratch_types=[
            pltpu.VMEM((HCH, K), jnp.int32),
            pltpu.VMEM((HCH, K), jnp.int32),
            pltpu.VMEM((HCH, K), jnp.int32),
            pltpu.VMEM((HCH, K), jnp.int32),
            pltpu.VMEM((K, D), jnp.float32),
            pltpu.VMEM((K, D), jnp.float32),
            pltpu.VMEM((K, D), jnp.float32),
            pltpu.SemaphoreType.DMA,
            pltpu.SemaphoreType.DMA,
            pltpu.SemaphoreType.DMA,
            pltpu.SemaphoreType.DMA,
            pltpu.VMEM_SHARED((NPAD, D), jnp.float32),
        ],
    )(y, src4, dst4)


def _final_body(a_ref, b_ref, dis_ref, out_ref):
    acc = a_ref[0] + b_ref[0]
    out_ref[...] = jnp.maximum(acc * dis_ref[:, 0:1], 0.0)


@jax.jit
def _final_kernel(acc, dis):
    return pl.pallas_call(
        _final_body,
        grid=(N_NODES // BR,),
        in_specs=[
            pl.BlockSpec((1, BR, D), lambda i: (0, i, 0)),
            pl.BlockSpec((1, BR, D), lambda i: (1, i, 0)),
            pl.BlockSpec((BR, DEGW), lambda i: (i, 0)),
        ],
        out_specs=pl.BlockSpec((BR, D), lambda i: (i, 0)),
        out_shape=jax.ShapeDtypeStruct((N_NODES, D), jnp.float32),
    )(acc, acc, dis)


@jax.jit
def kernel(x, edge_index, line_graph_val):
    src4 = edge_index[0].astype(jnp.int32).reshape(NW, NP, HCH, K)
    dst4 = edge_index[1].astype(jnp.int32).reshape(NW, NP, HCH, K)

    zeros_deg = jnp.zeros((NPAD, DEGW), jnp.float32)
    ones_k = jnp.ones((K, DEGW), jnp.float32)

    deg_p = _deg_kernel(dst4, zeros_deg, ones_k)
    y, dis = _scale_kernel(x, deg_p)
    acc = _agg_kernel(y, src4, dst4)
    return _final_kernel(acc, dis)


# speedup vs baseline: 1.2721x; 1.0183x over previous
import functools

import jax
import jax.numpy as jnp
from jax import lax
from jax.experimental import pallas as pl
from jax.experimental.pallas import tpu as pltpu
from jax.experimental.pallas import tpu_sc as plsc

N_NODES = 10000
N_EDGES = 320000
D = 128

NC = 2
NS = 16
NW = NC * NS
K = 80
CH = 125
NP = 5
HCH = CH // NP
NBUF = 3
NPAD = 10240
DEGW = 8
RPW = NPAD // NS
BR = 200

_mesh = lambda: plsc.VectorSubcoreMesh(core_axis_name="c", subcore_axis_name="s")


def _deg_body(dst_hbm, zeros_hbm, ones_hbm, deg_out, dst_v, ones_v, deg_sem,
              deg_sh):
    c = lax.axis_index("c")
    s = lax.axis_index("s")
    w = s * NC + c
    cp_idx = pltpu.async_copy(dst_hbm.at[w], dst_v, deg_sem)
    pltpu.sync_copy(ones_hbm, ones_v)
    pltpu.sync_copy(zeros_hbm.at[pl.ds(s * RPW, RPW)],
                    deg_sh.at[pl.ds(s * RPW, RPW)])
    cp_idx.wait()
    plsc.subcore_barrier()

    def issue(j, carry):
        p = j // HCH
        q = j % HCH
        pltpu.async_copy(ones_v, deg_sh.at[dst_v.at[p, q]], deg_sem, add=True)
        return carry

    lax.fori_loop(0, CH, issue, 0)

    def drain(j, carry):
        pltpu.make_async_copy(ones_v, deg_sh.at[dst_v.at[0, 0]],
                              deg_sem).wait()
        return carry

    lax.fori_loop(0, CH, drain, 0)
    plsc.subcore_barrier()
    pltpu.sync_copy(deg_sh.at[pl.ds(s * RPW, RPW)],
                    deg_out.at[c, pl.ds(s * RPW, RPW)])


@jax.jit
def _deg_kernel(dst4, zeros_deg, ones_k):
    return pl.kernel(
        _deg_body,
        out_type=jax.ShapeDtypeStruct((NC, NPAD, DEGW), jnp.float32),
        mesh=_mesh(),
        scratch_types=[
            pltpu.VMEM((NP, HCH, K), jnp.int32),
            pltpu.VMEM((K, DEGW), jnp.float32),
            pltpu.SemaphoreType.DMA,
            pltpu.VMEM_SHARED((NPAD, DEGW), jnp.float32),
        ],
    )(dst4, zeros_deg, ones_k)


def _scale_body(x_ref, dp_a_ref, dp_b_ref, y_ref, dis_ref):
    d = dp_a_ref[0] + dp_b_ref[0]
    r = lax.rsqrt(d)
    y_ref[...] = x_ref[...] * r[:, 0:1]
    dis_ref[...] = jnp.where(d > 0.0, r, 0.0)


@jax.jit
def _scale_kernel(x, deg_p):
    return pl.pallas_call(
        _scale_body,
        grid=(N_NODES // BR,),
        in_specs=[
            pl.BlockSpec((BR, D), lambda i: (i, 0)),
            pl.BlockSpec((1, BR, DEGW), lambda i: (0, i, 0)),
            pl.BlockSpec((1, BR, DEGW), lambda i: (1, i, 0)),
        ],
        out_specs=[
            pl.BlockSpec((BR, D), lambda i: (i, 0)),
            pl.BlockSpec((BR, DEGW), lambda i: (i, 0)),
        ],
        out_shape=[
            jax.ShapeDtypeStruct((N_NODES, D), jnp.float32),
            jax.ShapeDtypeStruct((N_NODES, DEGW), jnp.float32),
        ],
    )(x, deg_p, deg_p)


def _agg_body(y_hbm, src_hbm, dst_hbm, acc_out,
              src0, src1, dst0, dst1, rows0, rows1, rows2,
              g0, g1, g2, isem, acc_sh):
    c = lax.axis_index("c")
    s = lax.axis_index("s")
    w = s * NC + c
    rows = [rows0, rows1, rows2]
    gsem = [g0, g1, g2]
    srcb = [src0, src1]
    dstb = [dst0, dst1]

    cp_s = pltpu.async_copy(src_hbm.at[w, 0], src0, isem)
    cp_d = pltpu.async_copy(dst_hbm.at[w, 0], dst0, isem)

    def zrow(r, carry):
        for l in range(D // 16):
            rows1[r, pl.ds(l * 16, 16)] = jnp.zeros((16,), jnp.float32)
        return carry

    lax.fori_loop(0, K, zrow, 0)
    base = s * RPW
    for q in range(RPW // K):
        pltpu.sync_copy(rows1, acc_sh.at[pl.ds(base + q * K, K)])
    cp_s.wait()
    cp_d.wait()
    plsc.subcore_barrier()

    for j in range(NBUF):
        pltpu.async_copy(y_hbm.at[src0.at[j]], rows[j], gsem[j])

    TOT = NP * HCH
    for j in range(TOT):
        p, q, b = j // HCH, j % HCH, j % NBUF
        if q == 0 and p + 1 < NP:
            pltpu.async_copy(src_hbm.at[w, p + 1], srcb[(p + 1) % 2], isem)
            pltpu.async_copy(dst_hbm.at[w, p + 1], dstb[(p + 1) % 2], isem)
        if q == HCH - NBUF and p + 1 < NP:
            pltpu.make_async_copy(src_hbm.at[w, 0],
                                  srcb[(p + 1) % 2], isem).wait()
            pltpu.make_async_copy(dst_hbm.at[w, 0],
                                  dstb[(p + 1) % 2], isem).wait()
        pltpu.make_async_copy(y_hbm.at[src0.at[0]], rows[b], gsem[b]).wait()
        pltpu.sync_copy(rows[b], acc_sh.at[dstb[p % 2].at[q]], add=True)
        jn = j + NBUF
        if jn < TOT:
            pn, qn = jn // HCH, jn % HCH
            pltpu.async_copy(y_hbm.at[srcb[pn % 2].at[qn]], rows[b], gsem[b])

    plsc.subcore_barrier()
    pltpu.sync_copy(acc_sh.at[pl.ds(base, RPW)],
                    acc_out.at[c, pl.ds(base, RPW)])


@jax.jit
def _agg_kernel(y, src4, dst4):
    return pl.kernel(
        _agg_body,
        out_type=jax.ShapeDtypeStruct((NC, NPAD, D), jnp.float32),
        mesh=_mesh(),
        scratch_types=[
            pltpu.VMEM((HCH, K), jnp.int32),
            pltpu.VMEM((HCH, K), jnp.int32),
            pltpu.VMEM((HCH, K), jnp.int32),
            pltpu.VMEM((HCH, K), jnp.int32),
            pltpu.VMEM((K, D), jnp.float32),
            pltpu.VMEM((K, D), jnp.float32),
            pltpu.VMEM((K, D), jnp.float32),
            pltpu.SemaphoreType.DMA,
            pltpu.SemaphoreType.DMA,
            pltpu.SemaphoreType.DMA,
            pltpu.SemaphoreType.DMA,
            pltpu.VMEM_SHARED((NPAD, D), jnp.float32),
        ],
    )(y, src4, dst4)


def _final_body(a_ref, b_ref, dis_ref, out_ref):
    acc = a_ref[0] + b_ref[0]
    out_ref[...] = jnp.maximum(acc * dis_ref[:, 0:1], 0.0)


@jax.jit
def _final_kernel(acc, dis):
    return pl.pallas_call(
        _final_body,
        grid=(N_NODES // BR,),
        in_specs=[
            pl.BlockSpec((1, BR, D), lambda i: (0, i, 0)),
            pl.BlockSpec((1, BR, D), lambda i: (1, i, 0)),
            pl.BlockSpec((BR, DEGW), lambda i: (i, 0)),
        ],
        out_specs=pl.BlockSpec((BR, D), lambda i: (i, 0)),
        out_shape=jax.ShapeDtypeStruct((N_NODES, D), jnp.float32),
    )(acc, acc, dis)


@jax.jit
def kernel(x, edge_index, line_graph_val):
    src4 = edge_index[0].astype(jnp.int32).reshape(NW, NP, HCH, K)
    dst4 = edge_index[1].astype(jnp.int32).reshape(NW, NP, HCH, K)

    zeros_deg = jnp.zeros((NPAD, DEGW), jnp.float32)
    ones_k = jnp.ones((K, DEGW), jnp.float32)

    deg_p = _deg_kernel(dst4, zeros_deg, ones_k)
    y, dis = _scale_kernel(x, deg_p)
    acc = _agg_kernel(y, src4, dst4)
    return _final_kernel(acc, dis)


# speedup vs baseline: 1.5563x; 1.2233x over previous
import functools

import jax
import jax.numpy as jnp
from jax import lax
from jax.experimental import pallas as pl
from jax.experimental.pallas import tpu as pltpu
from jax.experimental.pallas import tpu_sc as plsc

N_NODES = 10000
N_EDGES = 320000
D = 128

NC = 2
NS = 16
NW = NC * NS
K = 80
CH = 125
NP = 5
HCH = CH // NP
NBUF = 3
NPAD = 10240
DEGW = 8
RPW = NPAD // NS
BR = 1000

_mesh = lambda: plsc.VectorSubcoreMesh(core_axis_name="c", subcore_axis_name="s")


def _deg_body(dst_hbm, zeros_hbm, ones_hbm, deg_out, dst_v, ones_v, deg_sem,
              deg_sh):
    c = lax.axis_index("c")
    s = lax.axis_index("s")
    w = s * NC + c
    cp_idx = pltpu.async_copy(dst_hbm.at[w], dst_v, deg_sem)
    pltpu.sync_copy(ones_hbm, ones_v)
    pltpu.sync_copy(zeros_hbm.at[pl.ds(s * RPW, RPW)],
                    deg_sh.at[pl.ds(s * RPW, RPW)])
    cp_idx.wait()
    plsc.subcore_barrier()

    def issue(j, carry):
        p = j // HCH
        q = j % HCH
        pltpu.async_copy(ones_v, deg_sh.at[dst_v.at[p, q]], deg_sem, add=True)
        return carry

    lax.fori_loop(0, CH, issue, 0)

    def drain(j, carry):
        pltpu.make_async_copy(ones_v, deg_sh.at[dst_v.at[0, 0]],
                              deg_sem).wait()
        return carry

    lax.fori_loop(0, CH, drain, 0)
    plsc.subcore_barrier()
    pltpu.sync_copy(deg_sh.at[pl.ds(s * RPW, RPW)],
                    deg_out.at[c, pl.ds(s * RPW, RPW)])


@jax.jit
def _deg_kernel(dst4, zeros_deg, ones_k):
    return pl.kernel(
        _deg_body,
        out_type=jax.ShapeDtypeStruct((NC, NPAD, DEGW), jnp.float32),
        mesh=_mesh(),
        scratch_types=[
            pltpu.VMEM((NP, HCH, K), jnp.int32),
            pltpu.VMEM((K, DEGW), jnp.float32),
            pltpu.SemaphoreType.DMA,
            pltpu.VMEM_SHARED((NPAD, DEGW), jnp.float32),
        ],
    )(dst4, zeros_deg, ones_k)


def _scale_body(x_ref, dp_a_ref, dp_b_ref, y_ref, dis_ref):
    d = dp_a_ref[0] + dp_b_ref[0]
    r = lax.rsqrt(d)
    y_ref[...] = x_ref[...] * r[:, 0:1]
    dis_ref[...] = jnp.where(d > 0.0, r, 0.0)


@jax.jit
def _scale_kernel(x, deg_p):
    return pl.pallas_call(
        _scale_body,
        grid=(N_NODES // BR,),
        in_specs=[
            pl.BlockSpec((BR, D), lambda i: (i, 0)),
            pl.BlockSpec((1, BR, DEGW), lambda i: (0, i, 0)),
            pl.BlockSpec((1, BR, DEGW), lambda i: (1, i, 0)),
        ],
        out_specs=[
            pl.BlockSpec((BR, D), lambda i: (i, 0)),
            pl.BlockSpec((BR, DEGW), lambda i: (i, 0)),
        ],
        out_shape=[
            jax.ShapeDtypeStruct((N_NODES, D), jnp.float32),
            jax.ShapeDtypeStruct((N_NODES, DEGW), jnp.float32),
        ],
    )(x, deg_p, deg_p)


def _agg_body(y_hbm, src_hbm, dst_hbm, acc_out,
              src0, src1, dst0, dst1, rows0, rows1, rows2,
              g0, g1, g2, isem, acc_sh):
    c = lax.axis_index("c")
    s = lax.axis_index("s")
    w = s * NC + c
    rows = [rows0, rows1, rows2]
    gsem = [g0, g1, g2]
    srcb = [src0, src1]
    dstb = [dst0, dst1]

    cp_s = pltpu.async_copy(src_hbm.at[w, 0], src0, isem)
    cp_d = pltpu.async_copy(dst_hbm.at[w, 0], dst0, isem)

    def zrow(r, carry):
        for l in range(D // 16):
            rows1[r, pl.ds(l * 16, 16)] = jnp.zeros((16,), jnp.float32)
        return carry

    lax.fori_loop(0, K, zrow, 0)
    base = s * RPW
    for q in range(RPW // K):
        pltpu.sync_copy(rows1, acc_sh.at[pl.ds(base + q * K, K)])
    cp_s.wait()
    cp_d.wait()
    plsc.subcore_barrier()

    for j in range(NBUF):
        pltpu.async_copy(y_hbm.at[src0.at[j]], rows[j], gsem[j])

    TOT = NP * HCH
    for j in range(TOT):
        p, q, b = j // HCH, j % HCH, j % NBUF
        if q == 0 and p + 1 < NP:
            pltpu.async_copy(src_hbm.at[w, p + 1], srcb[(p + 1) % 2], isem)
            pltpu.async_copy(dst_hbm.at[w, p + 1], dstb[(p + 1) % 2], isem)
        if q == HCH - NBUF and p + 1 < NP:
            pltpu.make_async_copy(src_hbm.at[w, 0],
                                  srcb[(p + 1) % 2], isem).wait()
            pltpu.make_async_copy(dst_hbm.at[w, 0],
                                  dstb[(p + 1) % 2], isem).wait()
        pltpu.make_async_copy(y_hbm.at[src0.at[0]], rows[b], gsem[b]).wait()
        pltpu.sync_copy(rows[b], acc_sh.at[dstb[p % 2].at[q]], add=True)
        jn = j + NBUF
        if jn < TOT:
            pn, qn = jn // HCH, jn % HCH
            pltpu.async_copy(y_hbm.at[srcb[pn % 2].at[qn]], rows[b], gsem[b])

    plsc.subcore_barrier()
    pltpu.sync_copy(acc_sh.at[pl.ds(base, RPW)],
                    acc_out.at[c, pl.ds(base, RPW)])


@jax.jit
def _agg_kernel(y, src4, dst4):
    return pl.kernel(
        _agg_body,
        out_type=jax.ShapeDtypeStruct((NC, NPAD, D), jnp.float32),
        mesh=_mesh(),
        scratch_types=[
            pltpu.VMEM((HCH, K), jnp.int32),
            pltpu.VMEM((HCH, K), jnp.int32),
            pltpu.VMEM((HCH, K), jnp.int32),
            pltpu.VMEM((HCH, K), jnp.int32),
            pltpu.VMEM((K, D), jnp.float32),
            pltpu.VMEM((K, D), jnp.float32),
            pltpu.VMEM((K, D), jnp.float32),
            pltpu.SemaphoreType.DMA,
            pltpu.SemaphoreType.DMA,
            pltpu.SemaphoreType.DMA,
            pltpu.SemaphoreType.DMA,
            pltpu.VMEM_SHARED((NPAD, D), jnp.float32),
        ],
    )(y, src4, dst4)


def _final_body(a_ref, b_ref, dis_ref, out_ref):
    acc = a_ref[0] + b_ref[0]
    out_ref[...] = jnp.maximum(acc * dis_ref[:, 0:1], 0.0)


@jax.jit
def _final_kernel(acc, dis):
    return pl.pallas_call(
        _final_body,
        grid=(N_NODES // BR,),
        in_specs=[
            pl.BlockSpec((1, BR, D), lambda i: (0, i, 0)),
            pl.BlockSpec((1, BR, D), lambda i: (1, i, 0)),
            pl.BlockSpec((BR, DEGW), lambda i: (i, 0)),
        ],
        out_specs=pl.BlockSpec((BR, D), lambda i: (i, 0)),
        out_shape=jax.ShapeDtypeStruct((N_NODES, D), jnp.float32),
    )(acc, acc, dis)


@jax.jit
def kernel(x, edge_index, line_graph_val):
    src4 = edge_index[0].astype(jnp.int32).reshape(NW, NP, HCH, K)
    dst4 = edge_index[1].astype(jnp.int32).reshape(NW, NP, HCH, K)

    zeros_deg = jnp.zeros((NPAD, DEGW), jnp.float32)
    ones_k = jnp.ones((K, DEGW), jnp.float32)

    deg_p = _deg_kernel(dst4, zeros_deg, ones_k)
    y, dis = _scale_kernel(x, deg_p)
    acc = _agg_kernel(y, src4, dst4)
    return _final_kernel(acc, dis)


# speedup vs baseline: 1.5935x; 1.0240x over previous
import functools

import jax
import jax.numpy as jnp
from jax import lax
from jax.experimental import pallas as pl
from jax.experimental.pallas import tpu as pltpu
from jax.experimental.pallas import tpu_sc as plsc

N_NODES = 10000
N_EDGES = 320000
D = 128

NC = 2
NS = 16
NW = NC * NS
K = 80
CH = 125
NP = 5
HCH = CH // NP
NBUF = 3
NPAD = 10240
DEGW = 8
RPW = NPAD // NS
BR = 2000

_mesh = lambda: plsc.VectorSubcoreMesh(core_axis_name="c", subcore_axis_name="s")


def _deg_body(dst_hbm, zeros_hbm, ones_hbm, deg_out, dst_v, ones_v, deg_sem,
              deg_sh):
    c = lax.axis_index("c")
    s = lax.axis_index("s")
    w = s * NC + c
    cp_idx = pltpu.async_copy(dst_hbm.at[w], dst_v, deg_sem)
    pltpu.sync_copy(ones_hbm, ones_v)
    pltpu.sync_copy(zeros_hbm.at[pl.ds(s * RPW, RPW)],
                    deg_sh.at[pl.ds(s * RPW, RPW)])
    cp_idx.wait()
    plsc.subcore_barrier()

    def issue(j, carry):
        p = j // HCH
        q = j % HCH
        pltpu.async_copy(ones_v, deg_sh.at[dst_v.at[p, q]], deg_sem, add=True)
        return carry

    lax.fori_loop(0, CH, issue, 0)

    def drain(j, carry):
        pltpu.make_async_copy(ones_v, deg_sh.at[dst_v.at[0, 0]],
                              deg_sem).wait()
        return carry

    lax.fori_loop(0, CH, drain, 0)
    plsc.subcore_barrier()
    pltpu.sync_copy(deg_sh.at[pl.ds(s * RPW, RPW)],
                    deg_out.at[c, pl.ds(s * RPW, RPW)])


@jax.jit
def _deg_kernel(dst4, zeros_deg, ones_k):
    return pl.kernel(
        _deg_body,
        out_type=jax.ShapeDtypeStruct((NC, NPAD, DEGW), jnp.float32),
        mesh=_mesh(),
        scratch_types=[
            pltpu.VMEM((NP, HCH, K), jnp.int32),
            pltpu.VMEM((K, DEGW), jnp.float32),
            pltpu.SemaphoreType.DMA,
            pltpu.VMEM_SHARED((NPAD, DEGW), jnp.float32),
        ],
    )(dst4, zeros_deg, ones_k)


def _scale_body(x_ref, dp_a_ref, dp_b_ref, y_ref, dis_ref):
    d = dp_a_ref[0] + dp_b_ref[0]
    r = lax.rsqrt(d)
    y_ref[...] = x_ref[...] * r[:, 0:1]
    dis_ref[...] = jnp.where(d > 0.0, r, 0.0)


@jax.jit
def _scale_kernel(x, deg_p):
    return pl.pallas_call(
        _scale_body,
        grid=(N_NODES // BR,),
        in_specs=[
            pl.BlockSpec((BR, D), lambda i: (i, 0)),
            pl.BlockSpec((1, BR, DEGW), lambda i: (0, i, 0)),
            pl.BlockSpec((1, BR, DEGW), lambda i: (1, i, 0)),
        ],
        out_specs=[
            pl.BlockSpec((BR, D), lambda i: (i, 0)),
            pl.BlockSpec((BR, DEGW), lambda i: (i, 0)),
        ],
        out_shape=[
            jax.ShapeDtypeStruct((N_NODES, D), jnp.float32),
            jax.ShapeDtypeStruct((N_NODES, DEGW), jnp.float32),
        ],
    )(x, deg_p, deg_p)


def _agg_body(y_hbm, src_hbm, dst_hbm, acc_out,
              src0, src1, dst0, dst1, rows0, rows1, rows2,
              g0, g1, g2, isem, acc_sh):
    c = lax.axis_index("c")
    s = lax.axis_index("s")
    w = s * NC + c
    rows = [rows0, rows1, rows2]
    gsem = [g0, g1, g2]
    srcb = [src0, src1]
    dstb = [dst0, dst1]

    cp_s = pltpu.async_copy(src_hbm.at[w, 0], src0, isem)
    cp_d = pltpu.async_copy(dst_hbm.at[w, 0], dst0, isem)

    def zrow(r, carry):
        for l in range(D // 16):
            rows1[r, pl.ds(l * 16, 16)] = jnp.zeros((16,), jnp.float32)
        return carry

    lax.fori_loop(0, K, zrow, 0)
    base = s * RPW
    for q in range(RPW // K):
        pltpu.sync_copy(rows1, acc_sh.at[pl.ds(base + q * K, K)])
    cp_s.wait()
    cp_d.wait()
    plsc.subcore_barrier()

    for j in range(NBUF):
        pltpu.async_copy(y_hbm.at[src0.at[j]], rows[j], gsem[j])

    TOT = NP * HCH
    for j in range(TOT):
        p, q, b = j // HCH, j % HCH, j % NBUF
        if q == 0 and p + 1 < NP:
            pltpu.async_copy(src_hbm.at[w, p + 1], srcb[(p + 1) % 2], isem)
            pltpu.async_copy(dst_hbm.at[w, p + 1], dstb[(p + 1) % 2], isem)
        if q == HCH - NBUF and p + 1 < NP:
            pltpu.make_async_copy(src_hbm.at[w, 0],
                                  srcb[(p + 1) % 2], isem).wait()
            pltpu.make_async_copy(dst_hbm.at[w, 0],
                                  dstb[(p + 1) % 2], isem).wait()
        pltpu.make_async_copy(y_hbm.at[src0.at[0]], rows[b], gsem[b]).wait()
        pltpu.sync_copy(rows[b], acc_sh.at[dstb[p % 2].at[q]], add=True)
        jn = j + NBUF
        if jn < TOT:
            pn, qn = jn // HCH, jn % HCH
            pltpu.async_copy(y_hbm.at[srcb[pn % 2].at[qn]], rows[b], gsem[b])

    plsc.subcore_barrier()
    pltpu.sync_copy(acc_sh.at[pl.ds(base, RPW)],
                    acc_out.at[c, pl.ds(base, RPW)])


@jax.jit
def _agg_kernel(y, src4, dst4):
    return pl.kernel(
        _agg_body,
        out_type=jax.ShapeDtypeStruct((NC, NPAD, D), jnp.float32),
        mesh=_mesh(),
        scratch_types=[
            pltpu.VMEM((HCH, K), jnp.int32),
            pltpu.VMEM((HCH, K), jnp.int32),
            pltpu.VMEM((HCH, K), jnp.int32),
            pltpu.VMEM((HCH, K), jnp.int32),
            pltpu.VMEM((K, D), jnp.float32),
            pltpu.VMEM((K, D), jnp.float32),
            pltpu.VMEM((K, D), jnp.float32),
            pltpu.SemaphoreType.DMA,
            pltpu.SemaphoreType.DMA,
            pltpu.SemaphoreType.DMA,
            pltpu.SemaphoreType.DMA,
            pltpu.VMEM_SHARED((NPAD, D), jnp.float32),
        ],
    )(y, src4, dst4)


def _final_body(a_ref, b_ref, dis_ref, out_ref):
    acc = a_ref[0] + b_ref[0]
    out_ref[...] = jnp.maximum(acc * dis_ref[:, 0:1], 0.0)


@jax.jit
def _final_kernel(acc, dis):
    return pl.pallas_call(
        _final_body,
        grid=(N_NODES // BR,),
        in_specs=[
            pl.BlockSpec((1, BR, D), lambda i: (0, i, 0)),
            pl.BlockSpec((1, BR, D), lambda i: (1, i, 0)),
            pl.BlockSpec((BR, DEGW), lambda i: (i, 0)),
        ],
        out_specs=pl.BlockSpec((BR, D), lambda i: (i, 0)),
        out_shape=jax.ShapeDtypeStruct((N_NODES, D), jnp.float32),
    )(acc, acc, dis)


@jax.jit
def kernel(x, edge_index, line_graph_val):
    src4 = edge_index[0].astype(jnp.int32).reshape(NW, NP, HCH, K)
    dst4 = edge_index[1].astype(jnp.int32).reshape(NW, NP, HCH, K)

    zeros_deg = jnp.zeros((NPAD, DEGW), jnp.float32)
    ones_k = jnp.ones((K, DEGW), jnp.float32)

    deg_p = _deg_kernel(dst4, zeros_deg, ones_k)
    y, dis = _scale_kernel(x, deg_p)
    acc = _agg_kernel(y, src4, dst4)
    return _final_kernel(acc, dis)
